# Initial kernel scaffold; baseline (speedup 1.0000x reference)
import jax
import jax.numpy as jnp
from jax.experimental import pallas as pl


def kernel(input_tensor, label_ids, pos_table, label_table, ln_gamma, ln_beta):
    raise NotImplementedError("write your pallas kernel here")


# speedup vs baseline: 2.2863x; 2.2863x over previous
import dataclasses

import jax
import jax.numpy as jnp
from jax import lax
from jax.experimental import pallas as pl
from jax.experimental.pallas import tpu as pltpu
from jax.experimental.pallas import tpu_sc as plsc

B, S, D = 4096, 200, 64
N = B * S
EPS = 1e-12
NC, NS = 2, 16
NW = NC * NS
TOK_W = N // NW
CHUNK = S
NCHUNK = TOK_W // CHUNK
L = 16
K = D // L


def _rsqrt16(v):
  i = plsc.bitcast(v, jnp.int32)
  i = jnp.int32(0x5F3759DF) - lax.shift_right_arithmetic(i, 1)
  r = plsc.bitcast(i, jnp.float32)
  for _ in range(3):
    r = r * (1.5 - 0.5 * v * r * r)
  return r


def _body(x_hbm, ids_hbm, pos_hbm, lab_hbm, gam_hbm, bet_hbm, out_hbm,
          xb, lb, ob, posv, idb, gv, bv, sem):
  wid = lax.axis_index("subcore") * NC + lax.axis_index("core")
  pltpu.sync_copy(pos_hbm, posv)
  pltpu.sync_copy(gam_hbm, gv)
  pltpu.sync_copy(bet_hbm, bv)
  g = [gv[pl.ds(k * L, L)] for k in range(K)]
  bt = [bv[pl.ds(k * L, L)] for k in range(K)]
  base0 = wid * TOK_W

  @pl.loop(0, NCHUNK)
  def _chunk(c):
    base = base0 + c * CHUNK
    pltpu.sync_copy(x_hbm.at[pl.ds(base, CHUNK)], xb)
    pltpu.sync_copy(ids_hbm.at[pl.ds(base, CHUNK)], idb)
    h = 104
    pltpu.async_copy(lab_hbm.at[idb.at[pl.ds(0, h)]],
                     lb.at[pl.ds(0, h)], sem).wait()
    pltpu.async_copy(lab_hbm.at[idb.at[pl.ds(h, CHUNK - h)]],
                     lb.at[pl.ds(h, CHUNK - h)], sem).wait()

    @plsc.parallel_loop(0, CHUNK, unroll=4)
    def _tok(t):
      e = []
      for k in range(K):
        sl = pl.ds(k * L, L)
        e.append(xb[t, sl] + posv[t, sl] + lb[t, sl])
      ssum = jnp.sum(e[0] + e[1] + e[2] + e[3])
      qsum = jnp.sum(e[0] * e[0] + e[1] * e[1] + e[2] * e[2] + e[3] * e[3])
      mean = ssum * (1.0 / D)
      var = qsum * (1.0 / D) - mean * mean
      r = _rsqrt16(jnp.broadcast_to(var + EPS, (L,)))
      for k in range(K):
        ob[t, pl.ds(k * L, L)] = (e[k] - mean) * r * g[k] + bt[k]

    pltpu.sync_copy(ob, out_hbm.at[pl.ds(base, CHUNK)])


@jax.jit
def kernel(input_tensor, label_ids, pos_table, label_table, ln_gamma, ln_beta):
  x2 = input_tensor.reshape(N, D)
  ids = label_ids.reshape(N).astype(jnp.int32)
  pos200 = pos_table[:S]
  mesh = plsc.VectorSubcoreMesh(core_axis_name="core",
                                subcore_axis_name="subcore")
  cp = pltpu.CompilerParams(needs_layout_passes=False,
                            use_tc_tiling_on_sc=False)
  run = pl.kernel(
      _body,
      out_type=jax.ShapeDtypeStruct((N, D), jnp.float32),
      mesh=mesh,
      scratch_types=[
          pltpu.VMEM((CHUNK, D), jnp.float32),
          pltpu.VMEM((CHUNK, D), jnp.float32),
          pltpu.VMEM((CHUNK, D), jnp.float32),
          pltpu.VMEM((S, D), jnp.float32),
          pltpu.VMEM((CHUNK,), jnp.int32),
          pltpu.VMEM((D,), jnp.float32),
          pltpu.VMEM((D,), jnp.float32),
          pltpu.SemaphoreType.DMA,
      ],
      compiler_params=cp,
  )
  out = run(x2, ids, pos200, label_table, ln_gamma, ln_beta)
  return out.reshape(B, S, D)


# speedup vs baseline: 2.2966x; 1.0045x over previous
import jax
import jax.numpy as jnp
from jax import lax
from jax.experimental import pallas as pl
from jax.experimental.pallas import tpu as pltpu
from jax.experimental.pallas import tpu_sc as plsc

B, S, D = 4096, 200, 64
N = B * S
EPS = 1e-12
NC, NS = 2, 16
NW = NC * NS
TOK_W = N // NW
CHUNK = S
NCHUNK = TOK_W // CHUNK
L = 16
K = D // L
H1 = 104
H2 = CHUNK - H1


def _rsqrt16(v):
  i = plsc.bitcast(v, jnp.int32)
  i = jnp.int32(0x5F3759DF) - lax.shift_right_arithmetic(i, 1)
  r = plsc.bitcast(i, jnp.float32)
  for _ in range(3):
    r = r * (1.5 - 0.5 * v * r * r)
  return r


def _body(x_hbm, ids_hbm, pos_hbm, lab_hbm, gam_hbm, bet_hbm, out_hbm,
          xb0, xb1, lb0, lb1, ob0, ob1, idb0, idb1, posv, gv, bv,
          sx0, sx1, si0, si1, sg0, sg1, so0, so1):
  xb = (xb0, xb1)
  lb = (lb0, lb1)
  ob = (ob0, ob1)
  idb = (idb0, idb1)
  sx = (sx0, sx1)
  si = (si0, si1)
  sg = (sg0, sg1)
  so = (so0, so1)

  wid = lax.axis_index("subcore") * NC + lax.axis_index("core")
  base0 = wid * TOK_W

  pltpu.sync_copy(pos_hbm, posv)
  pltpu.sync_copy(gam_hbm, gv)
  pltpu.sync_copy(bet_hbm, bv)
  g = [gv[pl.ds(k * L, L)] for k in range(K)]
  bt = [bv[pl.ds(k * L, L)] for k in range(K)]

  def issue_ids(c, p):
    pltpu.async_copy(ids_hbm.at[pl.ds(base0 + c * CHUNK, CHUNK)],
                     idb[p], si[p])

  def wait_ids(p):
    pltpu.make_async_copy(ids_hbm.at[pl.ds(0, CHUNK)], idb[p], si[p]).wait()

  def issue_x(c, p):
    pltpu.async_copy(x_hbm.at[pl.ds(base0 + c * CHUNK, CHUNK)], xb[p], sx[p])

  def wait_x(p):
    pltpu.make_async_copy(x_hbm.at[pl.ds(0, CHUNK)], xb[p], sx[p]).wait()

  def issue_gather(p):
    pltpu.async_copy(lab_hbm.at[idb[p].at[pl.ds(0, H1)]],
                     lb[p].at[pl.ds(0, H1)], sg[p])
    pltpu.async_copy(lab_hbm.at[idb[p].at[pl.ds(H1, H2)]],
                     lb[p].at[pl.ds(H1, H2)], sg[p])

  def wait_gather(p):
    pltpu.make_async_copy(lab_hbm.at[idb[p].at[pl.ds(0, H1)]],
                          lb[p].at[pl.ds(0, H1)], sg[p]).wait()
    pltpu.make_async_copy(lab_hbm.at[idb[p].at[pl.ds(H1, H2)]],
                          lb[p].at[pl.ds(H1, H2)], sg[p]).wait()

  def issue_out(c, p):
    pltpu.async_copy(ob[p], out_hbm.at[pl.ds(base0 + c * CHUNK, CHUNK)],
                     so[p])

  def wait_out(p):
    pltpu.make_async_copy(ob[p], out_hbm.at[pl.ds(0, CHUNK)], so[p]).wait()

  def compute(p):
    xp, lp, op = xb[p], lb[p], ob[p]

    @plsc.parallel_loop(0, CHUNK, unroll=4)
    def _tok(t):
      e = []
      for k in range(K):
        sl = pl.ds(k * L, L)
        e.append(xp[t, sl] + posv[t, sl] + lp[t, sl])
      ssum = jnp.sum(e[0] + e[1] + e[2] + e[3])
      qsum = jnp.sum(e[0] * e[0] + e[1] * e[1] + e[2] * e[2] + e[3] * e[3])
      mean = ssum * (1.0 / D)
      var = qsum * (1.0 / D) - mean * mean
      r = _rsqrt16(jnp.broadcast_to(var + EPS, (L,)))
      for k in range(K):
        op[t, pl.ds(k * L, L)] = (e[k] - mean) * r * g[k] + bt[k]

  def stage(c, p, *, first=False, no_next_gather=False, no_prefetch=False):
    q = 1 - p
    if not no_next_gather:
      wait_ids(q)
      issue_gather(q)
    wait_x(p)
    wait_gather(p)
    if not no_prefetch:
      issue_ids(c + 2, p)
    if not first:
      wait_out(p)
    compute(p)
    issue_out(c, p)
    if not no_prefetch:
      issue_x(c + 2, p)

  issue_ids(0, 0)
  issue_x(0, 0)
  issue_ids(1, 1)
  issue_x(1, 1)
  wait_ids(0)
  issue_gather(0)

  stage(0, 0, first=True)
  stage(1, 1, first=True)

  @pl.loop(1, NCHUNK // 2 - 1)
  def _pair(cc):
    stage(2 * cc, 0)
    stage(2 * cc + 1, 1)

  stage(NCHUNK - 2, 0, no_prefetch=True)
  stage(NCHUNK - 1, 1, no_next_gather=True, no_prefetch=True)
  wait_out(0)
  wait_out(1)


@jax.jit
def kernel(input_tensor, label_ids, pos_table, label_table, ln_gamma, ln_beta):
  x2 = input_tensor.reshape(N, D)
  ids = label_ids.reshape(N).astype(jnp.int32)
  pos200 = pos_table[:S]
  mesh = plsc.VectorSubcoreMesh(core_axis_name="core",
                                subcore_axis_name="subcore")
  cp = pltpu.CompilerParams(needs_layout_passes=False,
                            use_tc_tiling_on_sc=False)
  run = pl.kernel(
      _body,
      out_type=jax.ShapeDtypeStruct((N, D), jnp.float32),
      mesh=mesh,
      scratch_types=[
          pltpu.VMEM((CHUNK, D), jnp.float32),
          pltpu.VMEM((CHUNK, D), jnp.float32),
          pltpu.VMEM((CHUNK, D), jnp.float32),
          pltpu.VMEM((CHUNK, D), jnp.float32),
          pltpu.VMEM((CHUNK, D), jnp.float32),
          pltpu.VMEM((CHUNK, D), jnp.float32),
          pltpu.VMEM((CHUNK,), jnp.int32),
          pltpu.VMEM((CHUNK,), jnp.int32),
          pltpu.VMEM((S, D), jnp.float32),
          pltpu.VMEM((D,), jnp.float32),
          pltpu.VMEM((D,), jnp.float32),
          pltpu.SemaphoreType.DMA,
          pltpu.SemaphoreType.DMA,
          pltpu.SemaphoreType.DMA,
          pltpu.SemaphoreType.DMA,
          pltpu.SemaphoreType.DMA,
          pltpu.SemaphoreType.DMA,
          pltpu.SemaphoreType.DMA,
          pltpu.SemaphoreType.DMA,
      ],
      compiler_params=cp,
  )
  out = run(x2, ids, pos200, label_table, ln_gamma, ln_beta)
  return out.reshape(B, S, D)
